# Initial kernel scaffold; baseline (speedup 1.0000x reference)
import jax
import jax.numpy as jnp
from jax.experimental import pallas as pl


def kernel(w0, w1, w2, b0, b1, b2, edge_index, pw_W, pw_b, pb_W, pb_b, pos_embed, msg_W, msg_b, upd_W, upd_b, eW, eb, pe1_W, pe1_b, pe2_W, pe2_b, pn1_W, pn1_b, pn2_W, pn2_b, weight_scale, bias_scale):
    raise NotImplementedError("write your pallas kernel here")


# speedup vs baseline: 84.4611x; 84.4611x over previous
import functools

import jax
import jax.numpy as jnp
from jax.experimental import pallas as pl

_LAYOUT = (784, 512, 512, 10)
_N = sum(_LAYOUT)
_NP = 1824
_D = 64
_B = 2
_F32 = jnp.float32


def _node_pre_kernel(nb_ref, pe_ref, pbW_ref, pbb_ref, msgW_ref, msgb_ref,
                     eW_ref, eb_ref, pwW_ref, pwb_ref,
                     x_ref, a_ref, c_ref, d_ref, u_ref, p_ref):
    pbW = pbW_ref[...]
    pbb = pbb_ref[...]
    pe = pe_ref[...]
    msgW1 = msgW_ref[0:_D, :]
    msgW2 = msgW_ref[_D:2 * _D, :]
    eW1 = eW_ref[0:_D, :]
    eW2 = eW_ref[_D:2 * _D, :]
    eW3 = eW_ref[2 * _D:3 * _D, :]
    pwb = pwb_ref[...]
    pwW = pwW_ref[...]

    const_m = msgb_ref[...] + jnp.dot(pwb, msgW2, preferred_element_type=_F32)
    const_e = eb_ref[...] + jnp.dot(pwb, eW3, preferred_element_type=_F32)
    u_ref[...] = jnp.dot(pwW, msgW2, preferred_element_type=_F32)
    p_ref[...] = jnp.dot(pwW, eW3, preferred_element_type=_F32)

    for b in range(_B):
        nb = nb_ref[b]
        x = nb * pbW + pbb + pe
        x_ref[b] = x
        a_ref[b] = jnp.dot(x, msgW1, preferred_element_type=_F32) + const_m
        c_ref[b] = jnp.dot(x, eW1, preferred_element_type=_F32)
        d_ref[b] = jnp.dot(x, eW2, preferred_element_type=_F32) + const_e


def _edge_kernel(w_ref, a_ref, c_ref, d_ref, u_ref, p_ref,
                 pe1W_ref, pe1b_ref, pe2W_ref, pe2b_ref, ws_ref,
                 eo_ref, agg_ref, *, ta, cc):
    ai = pl.program_id(1)
    w3 = w_ref[0][:, :, None]
    u3 = u_ref[...].reshape(1, 1, _D)
    p3 = p_ref[...].reshape(1, 1, _D)
    asrc = a_ref[0][:, None, :]
    csrc = c_ref[0][:, None, :]
    ddst = d_ref[0][None, :, :]

    z = jnp.maximum(csrc + ddst + w3 * p3, 0.0)
    z2 = z.reshape(ta * cc, _D)
    h = jnp.maximum(
        jnp.dot(z2, pe1W_ref[...], preferred_element_type=_F32) + pe1b_ref[...],
        0.0)
    eo = jnp.dot(h, pe2W_ref[...], preferred_element_type=_F32) + pe2b_ref[...]
    eo_ref[0] = eo * ws_ref[...]

    m = jnp.maximum(asrc + w3 * u3, 0.0)
    part = jnp.sum(m, axis=0)

    @pl.when(ai == 0)
    def _():
        agg_ref[0] = part

    @pl.when(ai != 0)
    def _():
        agg_ref[0] += part


def _node_post_kernel(x_ref, agg_ref, updW_ref, updb_ref,
                      pn1W_ref, pn1b_ref, pn2W_ref, pn2b_ref, scale_ref,
                      nout_ref):
    updW = updW_ref[...]
    updb = updb_ref[...]
    pn1W = pn1W_ref[...]
    pn1b = pn1b_ref[...]
    pn2W = pn2W_ref[...]
    pn2b = pn2b_ref[...]
    scale = scale_ref[...]
    for b in range(_B):
        x = x_ref[b]
        agg = agg_ref[b]
        xn = jnp.maximum(
            x + jnp.dot(agg, updW, preferred_element_type=_F32) + updb, 0.0)
        h = jnp.maximum(
            jnp.dot(xn, pn1W, preferred_element_type=_F32) + pn1b, 0.0)
        nout = jnp.dot(h, pn2W, preferred_element_type=_F32) + pn2b
        nout_ref[b] = nout * scale


def _run_edge_block(wflat, a_l, c_l, d_l, u, p, pe1_W, pe1b, pe2_W, pe2b, ws,
                    ta):
    bb, aa, cc = wflat.shape
    n_a = aa // ta
    grid = (bb, n_a)
    kern = functools.partial(_edge_kernel, ta=ta, cc=cc)
    eo, agg = pl.pallas_call(
        kern,
        grid=grid,
        in_specs=[
            pl.BlockSpec((1, ta, cc), lambda b, i: (b, i, 0)),
            pl.BlockSpec((1, ta, _D), lambda b, i: (b, i, 0)),
            pl.BlockSpec((1, ta, _D), lambda b, i: (b, i, 0)),
            pl.BlockSpec((1, cc, _D), lambda b, i: (b, 0, 0)),
            pl.BlockSpec((1, _D), lambda b, i: (0, 0)),
            pl.BlockSpec((1, _D), lambda b, i: (0, 0)),
            pl.BlockSpec((_D, _D), lambda b, i: (0, 0)),
            pl.BlockSpec((1, _D), lambda b, i: (0, 0)),
            pl.BlockSpec((_D, 1), lambda b, i: (0, 0)),
            pl.BlockSpec((1, 1), lambda b, i: (0, 0)),
            pl.BlockSpec((1, 1), lambda b, i: (0, 0)),
        ],
        out_specs=[
            pl.BlockSpec((1, ta * cc, 1), lambda b, i: (b, i, 0)),
            pl.BlockSpec((1, cc, _D), lambda b, i: (b, 0, 0)),
        ],
        out_shape=[
            jax.ShapeDtypeStruct((bb, aa * cc, 1), _F32),
            jax.ShapeDtypeStruct((bb, cc, _D), _F32),
        ],
    )(wflat, a_l, c_l, d_l, u, p, pe1_W, pe1b, pe2_W, pe2b, ws)
    return eo, agg


def kernel(w0, w1, w2, b0, b1, b2, edge_index, pw_W, pw_b, pb_W, pb_b,
           pos_embed, msg_W, msg_b, upd_W, upd_b, eW, eb, pe1_W, pe1_b,
           pe2_W, pe2_b, pn1_W, pn1_b, pn2_W, pn2_b, weight_scale, bias_scale):
    del edge_index
    bb = w0.shape[0]
    pad_n = _NP - _N

    nb = jnp.concatenate(
        [jnp.zeros((bb, _LAYOUT[0], 1), _F32), b0, b1, b2,
         jnp.zeros((bb, pad_n, 1), _F32)], axis=1)
    pe_pad = jnp.pad(pos_embed, ((0, pad_n), (0, 0)))

    row = lambda v: v.reshape(1, -1)
    x, a, c, d, u, p = pl.pallas_call(
        _node_pre_kernel,
        out_shape=[
            jax.ShapeDtypeStruct((bb, _NP, _D), _F32),
            jax.ShapeDtypeStruct((bb, _NP, _D), _F32),
            jax.ShapeDtypeStruct((bb, _NP, _D), _F32),
            jax.ShapeDtypeStruct((bb, _NP, _D), _F32),
            jax.ShapeDtypeStruct((1, _D), _F32),
            jax.ShapeDtypeStruct((1, _D), _F32),
        ],
    )(nb, pe_pad, row(pb_W), row(pb_b), msg_W, row(msg_b), eW, row(eb),
      row(pw_W), row(pw_b))

    offs = [0, 784, 1296, 1808]
    pe1b = row(pe1_b)
    pe2b = pe2_b.reshape(1, 1)

    w2p = jnp.pad(w2.reshape(bb, 512, 10), ((0, 0), (0, 0), (0, 6)))

    blocks = [
        (w0.reshape(bb, 784, 512), 0, 1, 512, 16),
        (w1.reshape(bb, 512, 512), 1, 2, 512, 16),
        (w2p, 2, 3, 16, 128),
    ]
    eos, aggs = [], []
    for wflat, i, dst_l, cc, ta in blocks:
        a_l = a[:, offs[i]:offs[i] + wflat.shape[1]]
        c_l = c[:, offs[i]:offs[i] + wflat.shape[1]]
        d_l = d[:, offs[dst_l]:offs[dst_l] + cc]
        ws = weight_scale[i].reshape(1, 1)
        eo, agg = _run_edge_block(wflat, a_l, c_l, d_l, u, p, pe1_W, pe1b,
                                  pe2_W, pe2b, ws, ta)
        eos.append(eo)
        aggs.append(agg)

    agg_full = jnp.concatenate(
        [jnp.zeros((bb, 784, _D), _F32), aggs[0], aggs[1],
         aggs[2][:, :10], jnp.zeros((bb, pad_n, _D), _F32)], axis=1)

    ramp = jnp.arange(_NP, dtype=jnp.int32)[:, None]
    scale_vec = jnp.where(ramp < 512, bias_scale[0],
                          jnp.where(ramp < 1024, bias_scale[1],
                                    bias_scale[2])).astype(_F32)

    n_out = pl.pallas_call(
        _node_post_kernel,
        out_shape=jax.ShapeDtypeStruct((bb, _NP, 1), _F32),
    )(x, agg_full, upd_W, row(upd_b), pn1_W, row(pn1_b), pn2_W,
      pn2_b.reshape(1, 1), scale_vec)

    w_out0 = eos[0].reshape(bb, 784, 512, 1)
    w_out1 = eos[1].reshape(bb, 512, 512, 1)
    w_out2 = eos[2].reshape(bb, 512, 16, 1)[:, :, :10]
    b_out0 = n_out[:, 0:512]
    b_out1 = n_out[:, 512:1024]
    b_out2 = n_out[:, 1024:1034]
    return (w_out0, w_out1, w_out2, b_out0, b_out1, b_out2)
